# baseline (speedup 1.0000x reference)
import functools

import jax
import jax.numpy as jnp
from jax.experimental import pallas as pl
from jax.experimental.pallas import tpu as pltpu

BATCH = 4096
IN_DIM = 128
INTER_DIM = 128
VOCAB = 1000
R = 16


def _body(ids_ref, *refs):
    dw_refs = refs[:R]
    comp_ref, lw_ref, b_ref, out_ref = refs[R:]
    dw = jnp.concatenate([r[...] for r in dw_refs], axis=0)
    c = comp_ref[...]
    inter = jnp.tanh(jnp.sum(dw * c[:, :, None], axis=1))
    logits = jax.lax.dot_general(
        inter, lw_ref[...], (((1,), (1,)), ((), ())),
        preferred_element_type=jnp.float32)
    out_ref[...] = jax.nn.sigmoid(logits + b_ref[...])


@jax.jit
def kernel(vocab_ids, compressed, decoder_weights, linear_w, linear_b):
    grid = (BATCH // R,)

    def dw_index(i, ids, j):
        return (ids[i * R + j], 0, 0)

    in_specs = [
        pl.BlockSpec((1, IN_DIM, INTER_DIM), functools.partial(dw_index, j=j))
        for j in range(R)
    ]
    in_specs.append(pl.BlockSpec((R, IN_DIM), lambda i, ids: (i, 0)))
    in_specs.append(pl.BlockSpec((VOCAB, INTER_DIM), lambda i, ids: (0, 0)))
    in_specs.append(pl.BlockSpec((1, VOCAB), lambda i, ids: (0, 0)))

    out = pl.pallas_call(
        _body,
        grid_spec=pltpu.PrefetchScalarGridSpec(
            num_scalar_prefetch=1,
            grid=grid,
            in_specs=in_specs,
            out_specs=pl.BlockSpec((R, VOCAB), lambda i, ids: (i, 0)),
        ),
        out_shape=jax.ShapeDtypeStruct((BATCH, VOCAB), jnp.float32),
    )(vocab_ids, *([decoder_weights] * R), compressed, linear_w,
      linear_b.reshape(1, VOCAB))
    return out


# speedup vs baseline: 1.3889x; 1.3889x over previous
import functools

import jax
import jax.numpy as jnp
from jax.experimental import pallas as pl
from jax.experimental.pallas import tpu as pltpu

BATCH = 4096
IN_DIM = 128
INTER_DIM = 128
VOCAB = 1000
R = 32
RM = 512


def _matvec_body(ids_ref, *refs):
    dw_refs = refs[:R]
    comp_ref, out_ref = refs[R:]
    dw = jnp.concatenate([r[...] for r in dw_refs], axis=0)
    c = comp_ref[...]
    out_ref[...] = jnp.tanh(jnp.sum(dw * c[:, :, None], axis=1))


def _logits_body(inter_ref, lw_ref, b_ref, out_ref):
    logits = jax.lax.dot_general(
        inter_ref[...], lw_ref[...], (((1,), (1,)), ((), ())),
        preferred_element_type=jnp.float32)
    out_ref[...] = jax.nn.sigmoid(logits + b_ref[...])


@jax.jit
def kernel(vocab_ids, compressed, decoder_weights, linear_w, linear_b):
    def dw_index(i, ids, j):
        return (ids[i * R + j], 0, 0)

    in_specs = [
        pl.BlockSpec((1, IN_DIM, INTER_DIM), functools.partial(dw_index, j=j))
        for j in range(R)
    ]
    in_specs.append(pl.BlockSpec((R, IN_DIM), lambda i, ids: (i, 0)))

    inter = pl.pallas_call(
        _matvec_body,
        grid_spec=pltpu.PrefetchScalarGridSpec(
            num_scalar_prefetch=1,
            grid=(BATCH // R,),
            in_specs=in_specs,
            out_specs=pl.BlockSpec((R, INTER_DIM), lambda i, ids: (i, 0)),
        ),
        out_shape=jax.ShapeDtypeStruct((BATCH, INTER_DIM), jnp.float32),
    )(vocab_ids, *([decoder_weights] * R), compressed)

    out = pl.pallas_call(
        _logits_body,
        grid=(BATCH // RM,),
        in_specs=[
            pl.BlockSpec((RM, INTER_DIM), lambda i: (i, 0)),
            pl.BlockSpec((VOCAB, INTER_DIM), lambda i: (0, 0)),
            pl.BlockSpec((1, VOCAB), lambda i: (0, 0)),
        ],
        out_specs=pl.BlockSpec((RM, VOCAB), lambda i: (i, 0)),
        out_shape=jax.ShapeDtypeStruct((BATCH, VOCAB), jnp.float32),
    )(inter, linear_w, linear_b.reshape(1, VOCAB))
    return out
